# baseline (speedup 1.0000x reference)
import jax
import jax.numpy as jnp
from jax import lax
from jax.experimental import pallas as pl
from jax.experimental.pallas import tpu as pltpu


def _round_up(x, m):
    return (x + m - 1) // m * m


def _ent_agg_kernel(tail_ref, type_ref, head_ref, ent_ref, wt_ref, out_ref,
                    b0, b1, b2, b3):
    k = pl.program_id(0)
    tile_k = tail_ref.shape[1]
    channel = out_ref.shape[1]
    bufs = (b0, b1, b2, b3)
    unroll = 32

    @pl.when(k == 0)
    def _init():
        for b in bufs:
            b[...] = jnp.zeros_like(b)

    def chunk(ci, carry):
        base = ci * unroll
        for u in range(unroll):
            e = base + u
            t = tail_ref[0, e]
            r = type_ref[0, e]
            h = head_ref[0, e]
            row = ent_ref[pl.ds(t, 1), :] * wt_ref[pl.ds(r, 1), :]
            b = bufs[u % 4]
            b[pl.ds(h, 1), :] = b[pl.ds(h, 1), :] + row
        return carry

    lax.fori_loop(0, tile_k // unroll, chunk, 0)

    @pl.when(k == pl.num_programs(0) - 1)
    def _finalize():
        tot = (b0[...] + b1[...]) + (b2[...] + b3[...])
        n_out = out_ref.shape[0]
        cnt = tot[:n_out, channel:channel + 1]
        inv = pl.reciprocal(jnp.maximum(cnt, 1.0), approx=False)
        out_ref[...] = tot[:n_out, :channel] * inv


def _entity_aggregate(head, tail, type_m1, entity_emb, weight, *,
                      tile_k=2048):
    n_entities, channel = entity_emb.shape
    n_edges = head.shape[0]
    n_edge_pad = _round_up(n_edges, tile_k)
    n_ent_pad = _round_up(n_entities, 8)
    n_rel_pad = _round_up(weight.shape[0], 8)
    n_rows = n_ent_pad + 8

    pad_e = ((0, 0), (0, n_edge_pad - n_edges))
    tail_pad = jnp.pad(tail.reshape(1, -1), pad_e)
    type_pad = jnp.pad(type_m1.reshape(1, -1), pad_e)
    head_pad = jnp.pad(head.reshape(1, -1), pad_e,
                       constant_values=n_ent_pad)
    w_aug = _round_up(channel + 1, 128)
    ent_aug = jnp.pad(
        jnp.concatenate(
            [entity_emb, jnp.ones((n_entities, 1), jnp.float32)], axis=1),
        ((0, n_ent_pad - n_entities), (0, w_aug - channel - 1)))
    wt_aug = jnp.pad(
        jnp.concatenate(
            [weight, jnp.ones((weight.shape[0], 1), jnp.float32)], axis=1),
        ((0, n_rel_pad - weight.shape[0]), (0, w_aug - channel - 1)))

    out = pl.pallas_call(
        _ent_agg_kernel,
        out_shape=jax.ShapeDtypeStruct((n_ent_pad, channel), jnp.float32),
        grid=(n_edge_pad // tile_k,),
        in_specs=[
            pl.BlockSpec((1, tile_k), lambda k: (0, k),
                         memory_space=pltpu.SMEM),
            pl.BlockSpec((1, tile_k), lambda k: (0, k),
                         memory_space=pltpu.SMEM),
            pl.BlockSpec((1, tile_k), lambda k: (0, k),
                         memory_space=pltpu.SMEM),
            pl.BlockSpec((n_ent_pad, w_aug), lambda k: (0, 0)),
            pl.BlockSpec((n_rel_pad, w_aug), lambda k: (0, 0)),
        ],
        out_specs=pl.BlockSpec((n_ent_pad, channel), lambda k: (0, 0)),
        scratch_shapes=[pltpu.VMEM((n_rows, w_aug), jnp.float32)
                        for _ in range(4)],
        compiler_params=pltpu.CompilerParams(
            dimension_semantics=("arbitrary",)),
    )(tail_pad, type_pad, head_pad, ent_aug, wt_aug)
    return out[:n_entities]


def _user_agg_kernel(user_ref, latent_ref, dw_ref, inter_ref, ent_ref,
                     out_ref):
    ua = jnp.dot(inter_ref[...].astype(jnp.bfloat16), ent_ref[...],
                 preferred_element_type=jnp.float32)
    s = lax.dot_general(user_ref[...], latent_ref[...],
                        (((1,), (1,)), ((), ())),
                        preferred_element_type=jnp.float32)
    s = s - jnp.max(s, axis=1, keepdims=True)
    e = jnp.exp(s)
    score = e * pl.reciprocal(jnp.sum(e, axis=1, keepdims=True),
                              approx=False)
    gate = jnp.dot(score, dw_ref[...],
                   preferred_element_type=jnp.float32)
    out_ref[...] = ua * (gate + 1.0)


def _user_aggregate(user_emb, latent_emb, interact_dense, ent_bf16, dw,
                    *, tile_u=256):
    n_users, channel = user_emb.shape
    n_ent_pad = ent_bf16.shape[0]
    n_factors = latent_emb.shape[0]
    tile_u = min(tile_u, _round_up(n_users, 8))
    n_users_pad = _round_up(n_users, tile_u)

    user_pad = jnp.pad(user_emb, ((0, n_users_pad - n_users), (0, 0)))
    inter_pad = jnp.pad(interact_dense,
                        ((0, n_users_pad - n_users),
                         (0, n_ent_pad - interact_dense.shape[1])))

    out = pl.pallas_call(
        _user_agg_kernel,
        out_shape=jax.ShapeDtypeStruct((n_users_pad, channel), jnp.float32),
        grid=(n_users_pad // tile_u,),
        in_specs=[
            pl.BlockSpec((tile_u, channel), lambda i: (i, 0)),
            pl.BlockSpec((n_factors, channel), lambda i: (0, 0)),
            pl.BlockSpec((n_factors, channel), lambda i: (0, 0)),
            pl.BlockSpec((tile_u, n_ent_pad), lambda i: (i, 0)),
            pl.BlockSpec((n_ent_pad, channel), lambda i: (0, 0)),
        ],
        out_specs=pl.BlockSpec((tile_u, channel), lambda i: (i, 0)),
        compiler_params=pltpu.CompilerParams(
            dimension_semantics=("parallel",)),
    )(user_pad, latent_emb, dw, inter_pad, ent_bf16)
    return out[:n_users]


def kernel(entity_emb, user_emb, latent_emb, edge_index, edge_type,
           interact_dense, weight, disen_weight_att):
    n_entities, channel = entity_emb.shape
    head = edge_index[0].astype(jnp.int32)
    tail = edge_index[1].astype(jnp.int32)
    type_m1 = (edge_type - 1).astype(jnp.int32)

    entity_agg = _entity_aggregate(head, tail, type_m1, entity_emb, weight)

    dw = jax.nn.softmax(disen_weight_att, axis=-1) @ weight
    n_ent_pad = _round_up(n_entities, 8)
    ent_bf16 = jnp.pad(entity_emb,
                       ((0, n_ent_pad - n_entities), (0, 0))).astype(
                           jnp.bfloat16)
    user_agg = _user_aggregate(user_emb, latent_emb, interact_dense,
                               ent_bf16, dw)
    return entity_agg, user_agg


# speedup vs baseline: 1.0123x; 1.0123x over previous
import jax
import jax.numpy as jnp
from jax import lax
from jax.experimental import pallas as pl
from jax.experimental.pallas import tpu as pltpu


def _round_up(x, m):
    return (x + m - 1) // m * m


def _ent_agg_kernel(tail_ref, type_ref, head_ref, ent_ref, wt_ref, out_ref,
                    b0, b1, b2, b3):
    k = pl.program_id(0)
    tile_k = tail_ref.shape[1]
    channel = out_ref.shape[1]
    bufs = (b0, b1, b2, b3)
    unroll = 64

    @pl.when(k == 0)
    def _init():
        for b in bufs:
            b[...] = jnp.zeros_like(b)

    def chunk(ci, carry):
        base = ci * unroll
        for u in range(unroll):
            e = base + u
            t = tail_ref[0, e]
            r = type_ref[0, e]
            h = head_ref[0, e]
            row = ent_ref[pl.ds(t, 1), :] * wt_ref[pl.ds(r, 1), :]
            b = bufs[u % 4]
            b[pl.ds(h, 1), :] = b[pl.ds(h, 1), :] + row
        return carry

    lax.fori_loop(0, tile_k // unroll, chunk, 0)

    @pl.when(k == pl.num_programs(0) - 1)
    def _finalize():
        tot = (b0[...] + b1[...]) + (b2[...] + b3[...])
        n_out = out_ref.shape[0]
        cnt = tot[:n_out, channel:channel + 1]
        inv = pl.reciprocal(jnp.maximum(cnt, 1.0), approx=False)
        out_ref[...] = tot[:n_out, :channel] * inv


def _entity_aggregate(head, tail, type_m1, entity_emb, weight, *,
                      tile_k=2048):
    n_entities, channel = entity_emb.shape
    n_edges = head.shape[0]
    n_edge_pad = _round_up(n_edges, tile_k)
    n_ent_pad = _round_up(n_entities, 8)
    n_rel_pad = _round_up(weight.shape[0], 8)
    n_rows = n_ent_pad + 8

    pad_e = ((0, 0), (0, n_edge_pad - n_edges))
    tail_pad = jnp.pad(tail.reshape(1, -1), pad_e)
    type_pad = jnp.pad(type_m1.reshape(1, -1), pad_e)
    head_pad = jnp.pad(head.reshape(1, -1), pad_e,
                       constant_values=n_ent_pad)
    w_aug = _round_up(channel + 1, 128)
    ent_aug = jnp.pad(
        jnp.concatenate(
            [entity_emb, jnp.ones((n_entities, 1), jnp.float32)], axis=1),
        ((0, n_ent_pad - n_entities), (0, w_aug - channel - 1)))
    wt_aug = jnp.pad(
        jnp.concatenate(
            [weight, jnp.ones((weight.shape[0], 1), jnp.float32)], axis=1),
        ((0, n_rel_pad - weight.shape[0]), (0, w_aug - channel - 1)))

    out = pl.pallas_call(
        _ent_agg_kernel,
        out_shape=jax.ShapeDtypeStruct((n_ent_pad, channel), jnp.float32),
        grid=(n_edge_pad // tile_k,),
        in_specs=[
            pl.BlockSpec((1, tile_k), lambda k: (0, k),
                         memory_space=pltpu.SMEM),
            pl.BlockSpec((1, tile_k), lambda k: (0, k),
                         memory_space=pltpu.SMEM),
            pl.BlockSpec((1, tile_k), lambda k: (0, k),
                         memory_space=pltpu.SMEM),
            pl.BlockSpec((n_ent_pad, w_aug), lambda k: (0, 0)),
            pl.BlockSpec((n_rel_pad, w_aug), lambda k: (0, 0)),
        ],
        out_specs=pl.BlockSpec((n_ent_pad, channel), lambda k: (0, 0)),
        scratch_shapes=[pltpu.VMEM((n_rows, w_aug), jnp.float32)
                        for _ in range(4)],
        compiler_params=pltpu.CompilerParams(
            dimension_semantics=("arbitrary",)),
    )(tail_pad, type_pad, head_pad, ent_aug, wt_aug)
    return out[:n_entities]


def _user_agg_kernel(user_ref, latent_ref, dw_ref, inter_ref, ent_ref,
                     out_ref):
    ua = jnp.dot(inter_ref[...].astype(jnp.bfloat16), ent_ref[...],
                 preferred_element_type=jnp.float32)
    s = lax.dot_general(user_ref[...], latent_ref[...],
                        (((1,), (1,)), ((), ())),
                        preferred_element_type=jnp.float32)
    s = s - jnp.max(s, axis=1, keepdims=True)
    e = jnp.exp(s)
    score = e * pl.reciprocal(jnp.sum(e, axis=1, keepdims=True),
                              approx=False)
    gate = jnp.dot(score, dw_ref[...],
                   preferred_element_type=jnp.float32)
    out_ref[...] = ua * (gate + 1.0)


def _user_aggregate(user_emb, latent_emb, interact_dense, ent_bf16, dw,
                    *, tile_u=256):
    n_users, channel = user_emb.shape
    n_ent_pad = ent_bf16.shape[0]
    n_factors = latent_emb.shape[0]
    tile_u = min(tile_u, _round_up(n_users, 8))
    n_users_pad = _round_up(n_users, tile_u)

    user_pad = jnp.pad(user_emb, ((0, n_users_pad - n_users), (0, 0)))
    inter_pad = jnp.pad(interact_dense,
                        ((0, n_users_pad - n_users),
                         (0, n_ent_pad - interact_dense.shape[1])))

    out = pl.pallas_call(
        _user_agg_kernel,
        out_shape=jax.ShapeDtypeStruct((n_users_pad, channel), jnp.float32),
        grid=(n_users_pad // tile_u,),
        in_specs=[
            pl.BlockSpec((tile_u, channel), lambda i: (i, 0)),
            pl.BlockSpec((n_factors, channel), lambda i: (0, 0)),
            pl.BlockSpec((n_factors, channel), lambda i: (0, 0)),
            pl.BlockSpec((tile_u, n_ent_pad), lambda i: (i, 0)),
            pl.BlockSpec((n_ent_pad, channel), lambda i: (0, 0)),
        ],
        out_specs=pl.BlockSpec((tile_u, channel), lambda i: (i, 0)),
        compiler_params=pltpu.CompilerParams(
            dimension_semantics=("parallel",)),
    )(user_pad, latent_emb, dw, inter_pad, ent_bf16)
    return out[:n_users]


def kernel(entity_emb, user_emb, latent_emb, edge_index, edge_type,
           interact_dense, weight, disen_weight_att):
    n_entities, channel = entity_emb.shape
    head = edge_index[0].astype(jnp.int32)
    tail = edge_index[1].astype(jnp.int32)
    type_m1 = (edge_type - 1).astype(jnp.int32)

    entity_agg = _entity_aggregate(head, tail, type_m1, entity_emb, weight)

    dw = jax.nn.softmax(disen_weight_att, axis=-1) @ weight
    n_ent_pad = _round_up(n_entities, 8)
    ent_bf16 = jnp.pad(entity_emb,
                       ((0, n_ent_pad - n_entities), (0, 0))).astype(
                           jnp.bfloat16)
    user_agg = _user_aggregate(user_emb, latent_emb, interact_dense,
                               ent_bf16, dw)
    return entity_agg, user_agg


# speedup vs baseline: 2.0409x; 2.0160x over previous
import jax
import jax.numpy as jnp
from jax import lax
from jax.experimental import pallas as pl
from jax.experimental.pallas import tpu as pltpu


def _round_up(x, m):
    return (x + m - 1) // m * m


def _ent_agg_kernel(tail_ref, type_ref, head_ref, ent_ref, wt_ref, out_ref,
                    b0, b1, b2, b3, c0, c1, c2, c3):
    k = pl.program_id(0)
    tile_k = tail_ref.shape[1]
    channel = out_ref.shape[1]
    bufs = (b0, b1, b2, b3)
    cnts = (c0, c1, c2, c3)
    unroll = 64

    @pl.when(k == 0)
    def _init():
        for b in bufs + cnts:
            b[...] = jnp.zeros_like(b)

    def chunk(ci, carry):
        base = ci * unroll
        for u in range(unroll):
            e = base + u
            t = tail_ref[0, e]
            r = type_ref[0, e]
            h = head_ref[0, e]
            row = ent_ref[pl.ds(t, 1), :] * wt_ref[pl.ds(r, 1), :]
            b = bufs[u % 4]
            c = cnts[u % 4]
            b[pl.ds(h, 1), :] = b[pl.ds(h, 1), :] + row
            c[pl.ds(h, 1), :] = c[pl.ds(h, 1), :] + 1.0
        return carry

    lax.fori_loop(0, tile_k // unroll, chunk, 0)

    @pl.when(k == pl.num_programs(0) - 1)
    def _finalize():
        tot = (b0[...] + b1[...]) + (b2[...] + b3[...])
        ctot = (c0[...] + c1[...]) + (c2[...] + c3[...])
        n_out = out_ref.shape[0]
        cnt = ctot[:n_out, 0:1]
        inv = pl.reciprocal(jnp.maximum(cnt, 1.0), approx=False)
        out_ref[...] = tot[:n_out, :] * inv


def _entity_aggregate(head, tail, type_m1, entity_emb, weight, *,
                      tile_k=2048):
    n_entities, channel = entity_emb.shape
    n_edges = head.shape[0]
    n_edge_pad = _round_up(n_edges, tile_k)
    n_ent_pad = _round_up(n_entities, 8)
    n_rel_pad = _round_up(weight.shape[0], 8)
    n_rows = n_ent_pad + 8

    pad_e = ((0, 0), (0, n_edge_pad - n_edges))
    tail_pad = jnp.pad(tail.reshape(1, -1), pad_e)
    type_pad = jnp.pad(type_m1.reshape(1, -1), pad_e)
    head_pad = jnp.pad(head.reshape(1, -1), pad_e,
                       constant_values=n_ent_pad)
    ent_pad = jnp.pad(entity_emb, ((0, n_ent_pad - n_entities), (0, 0)))
    wt_pad = jnp.pad(weight, ((0, n_rel_pad - weight.shape[0]), (0, 0)))

    out = pl.pallas_call(
        _ent_agg_kernel,
        out_shape=jax.ShapeDtypeStruct((n_ent_pad, channel), jnp.float32),
        grid=(n_edge_pad // tile_k,),
        in_specs=[
            pl.BlockSpec((1, tile_k), lambda k: (0, k),
                         memory_space=pltpu.SMEM),
            pl.BlockSpec((1, tile_k), lambda k: (0, k),
                         memory_space=pltpu.SMEM),
            pl.BlockSpec((1, tile_k), lambda k: (0, k),
                         memory_space=pltpu.SMEM),
            pl.BlockSpec((n_ent_pad, channel), lambda k: (0, 0)),
            pl.BlockSpec((n_rel_pad, channel), lambda k: (0, 0)),
        ],
        out_specs=pl.BlockSpec((n_ent_pad, channel), lambda k: (0, 0)),
        scratch_shapes=([pltpu.VMEM((n_rows, channel), jnp.float32)
                         for _ in range(4)] +
                        [pltpu.VMEM((n_rows, 8), jnp.float32)
                         for _ in range(4)]),
        compiler_params=pltpu.CompilerParams(
            dimension_semantics=("arbitrary",)),
    )(tail_pad, type_pad, head_pad, ent_pad, wt_pad)
    return out[:n_entities]


def _user_agg_kernel(user_ref, latent_ref, dw_ref, inter_ref, ent_ref,
                     out_ref):
    ua = jnp.dot(inter_ref[...].astype(jnp.bfloat16), ent_ref[...],
                 preferred_element_type=jnp.float32)
    s = lax.dot_general(user_ref[...], latent_ref[...],
                        (((1,), (1,)), ((), ())),
                        preferred_element_type=jnp.float32)
    s = s - jnp.max(s, axis=1, keepdims=True)
    e = jnp.exp(s)
    score = e * pl.reciprocal(jnp.sum(e, axis=1, keepdims=True),
                              approx=False)
    gate = jnp.dot(score, dw_ref[...],
                   preferred_element_type=jnp.float32)
    out_ref[...] = ua * (gate + 1.0)


def _user_aggregate(user_emb, latent_emb, interact_dense, ent_bf16, dw,
                    *, tile_u=256):
    n_users, channel = user_emb.shape
    n_ent_pad = ent_bf16.shape[0]
    n_factors = latent_emb.shape[0]
    tile_u = min(tile_u, _round_up(n_users, 8))
    n_users_pad = _round_up(n_users, tile_u)

    user_pad = jnp.pad(user_emb, ((0, n_users_pad - n_users), (0, 0)))
    inter_pad = jnp.pad(interact_dense,
                        ((0, n_users_pad - n_users),
                         (0, n_ent_pad - interact_dense.shape[1])))

    out = pl.pallas_call(
        _user_agg_kernel,
        out_shape=jax.ShapeDtypeStruct((n_users_pad, channel), jnp.float32),
        grid=(n_users_pad // tile_u,),
        in_specs=[
            pl.BlockSpec((tile_u, channel), lambda i: (i, 0)),
            pl.BlockSpec((n_factors, channel), lambda i: (0, 0)),
            pl.BlockSpec((n_factors, channel), lambda i: (0, 0)),
            pl.BlockSpec((tile_u, n_ent_pad), lambda i: (i, 0)),
            pl.BlockSpec((n_ent_pad, channel), lambda i: (0, 0)),
        ],
        out_specs=pl.BlockSpec((tile_u, channel), lambda i: (i, 0)),
        compiler_params=pltpu.CompilerParams(
            dimension_semantics=("parallel",)),
    )(user_pad, latent_emb, dw, inter_pad, ent_bf16)
    return out[:n_users]


def kernel(entity_emb, user_emb, latent_emb, edge_index, edge_type,
           interact_dense, weight, disen_weight_att):
    n_entities, channel = entity_emb.shape
    head = edge_index[0].astype(jnp.int32)
    tail = edge_index[1].astype(jnp.int32)
    type_m1 = (edge_type - 1).astype(jnp.int32)

    entity_agg = _entity_aggregate(head, tail, type_m1, entity_emb, weight)

    dw = jax.nn.softmax(disen_weight_att, axis=-1) @ weight
    n_ent_pad = _round_up(n_entities, 8)
    ent_bf16 = jnp.pad(entity_emb,
                       ((0, n_ent_pad - n_entities), (0, 0))).astype(
                           jnp.bfloat16)
    user_agg = _user_aggregate(user_emb, latent_emb, interact_dense,
                               ent_bf16, dw)
    return entity_agg, user_agg
